# baseline (device time: 165200 ns/iter reference)
import numpy as np
import jax
import jax.numpy as jnp
from jax import lax
from jax.experimental import pallas as pl
from jax.experimental.pallas import tpu as pltpu

N_DEV = 8
B, SQ, D = 1, 1024, 1024
HQ, DH = 8, 128
CHUNK = SQ // N_DEV
SCALE = 0.08838834764831843


def _rope_consts():
    inv = 1.0 / (10000.0 ** (np.arange(0, DH, 2) / DH))
    pos = np.arange(SQ)[:, None] * inv[None, :]
    cos = np.repeat(np.cos(pos), 2, axis=-1)
    sin = np.repeat(np.sin(pos), 2, axis=-1)
    cosf = np.tile(cos, (1, HQ)).astype(np.float32)
    sinf = np.tile(sin, (1, HQ)).astype(np.float32)
    p1 = np.zeros((DH, DH), np.float32)
    for m in range(DH // 2):
        p1[2 * m + 1, 2 * m] = -1.0
        p1[2 * m, 2 * m + 1] = 1.0
    pmat = np.kron(np.eye(HQ, dtype=np.float32), p1)
    return cosf, sinf, pmat


_COSF, _SINF, _PMAT = _rope_consts()


def kernel(x, Wq, Wk, Wv, Wo):
    xb = x.reshape(SQ, D).astype(jnp.bfloat16)
    wq = Wq.astype(jnp.bfloat16)
    wk = Wk.astype(jnp.bfloat16)
    wv = Wv.astype(jnp.bfloat16)
    wo = Wo.astype(jnp.bfloat16)
    cosf = jnp.asarray(_COSF)
    sinf = jnp.asarray(_SINF)
    pmat = jnp.asarray(_PMAT, dtype=jnp.bfloat16)

    def body(x_ref, wq_ref, wk_ref, wv_ref, wo_ref, cos_ref, sin_ref, p_ref,
             out_ref, rs_buf, ag_buf, send_sems, recv_sems):
        my = lax.axis_index("i")
        right = lax.rem(my + 1, N_DEV)

        xv = x_ref[...]
        p = p_ref[...]

        def proj_rope(w_ref):
            t = jnp.dot(xv, w_ref[...], preferred_element_type=jnp.float32)
            tr = jnp.dot(t.astype(jnp.bfloat16), p,
                         preferred_element_type=jnp.float32)
            return (t * cos_ref[...] + tr * sin_ref[...]).astype(jnp.bfloat16)

        q = proj_rope(wq_ref)
        k = proj_rope(wk_ref)
        v = jnp.dot(xv, wv_ref[...],
                    preferred_element_type=jnp.float32).astype(jnp.bfloat16)

        acc = jnp.zeros((SQ, D), jnp.float32)
        for h in range(HQ):
            sl = slice(h * DH, (h + 1) * DH)
            s = lax.dot_general(q[:, sl], k[:, sl],
                                (((1,), (1,)), ((), ())),
                                preferred_element_type=jnp.float32) * SCALE
            m = jnp.max(s, axis=-1, keepdims=True)
            w = jnp.exp(s - m)
            w = w / jnp.sum(w, axis=-1, keepdims=True)
            ctx = jnp.dot(w.astype(jnp.bfloat16), v[:, sl],
                          preferred_element_type=jnp.float32)
            acc = acc + jnp.dot(ctx.astype(jnp.bfloat16), wo_ref[sl, :],
                                preferred_element_type=jnp.float32)
        out_ref[0] = acc

        for t in range(N_DEV - 1):
            cs = lax.rem(my - t + N_DEV, N_DEV)
            cr = lax.rem(my - t - 1 + 2 * N_DEV, N_DEV)
            rdma = pltpu.make_async_remote_copy(
                src_ref=out_ref.at[0, pl.ds(cs * CHUNK, CHUNK), :],
                dst_ref=rs_buf.at[t],
                send_sem=send_sems.at[t],
                recv_sem=recv_sems.at[t],
                device_id=(right,),
                device_id_type=pl.DeviceIdType.MESH,
            )
            rdma.start()
            rdma.wait()
            out_ref[0, pl.ds(cr * CHUNK, CHUNK), :] = (
                out_ref[0, pl.ds(cr * CHUNK, CHUNK), :] + rs_buf[t]
            )

        own = lax.rem(my + 1, N_DEV)
        for t in range(N_DEV - 1):
            src = (out_ref.at[0, pl.ds(own * CHUNK, CHUNK), :] if t == 0
                   else ag_buf.at[t - 1])
            rdma = pltpu.make_async_remote_copy(
                src_ref=src,
                dst_ref=ag_buf.at[t],
                send_sem=send_sems.at[N_DEV - 1 + t],
                recv_sem=recv_sems.at[N_DEV - 1 + t],
                device_id=(right,),
                device_id_type=pl.DeviceIdType.MESH,
            )
            rdma.start()
            rdma.wait()
            cr = lax.rem(my - t + N_DEV, N_DEV)
            out_ref[0, pl.ds(cr * CHUNK, CHUNK), :] = ag_buf[t]

    return pl.pallas_call(
        body,
        out_shape=jax.ShapeDtypeStruct((B, SQ, D), jnp.float32),
        in_specs=[pl.BlockSpec(memory_space=pltpu.VMEM)] * 8,
        out_specs=pl.BlockSpec(memory_space=pltpu.VMEM),
        scratch_shapes=[
            pltpu.VMEM((N_DEV - 1, CHUNK, D), jnp.float32),
            pltpu.VMEM((N_DEV - 1, CHUNK, D), jnp.float32),
            pltpu.SemaphoreType.DMA((2 * (N_DEV - 1),)),
            pltpu.SemaphoreType.DMA((2 * (N_DEV - 1),)),
        ],
    )(xb, wq, wk, wv, wo, cosf, sinf, pmat)


# device time: 110720 ns/iter; 1.4921x vs baseline; 1.4921x over previous
import numpy as np
import jax
import jax.numpy as jnp
from jax import lax
from jax.experimental import pallas as pl
from jax.experimental.pallas import tpu as pltpu

N_DEV = 8
B, SQ, D = 1, 1024, 1024
HQ, DH = 8, 128
CHUNK = SQ // N_DEV
SCALE = 0.08838834764831843


def _rope_consts():
    inv = 1.0 / (10000.0 ** (np.arange(0, DH, 2) / DH))
    pos = np.arange(SQ)[:, None] * inv[None, :]
    cos = np.repeat(np.cos(pos), 2, axis=-1)
    sin = np.repeat(np.sin(pos), 2, axis=-1)
    cosf = np.tile(cos, (1, HQ)).astype(np.float32)
    sinf = np.tile(sin, (1, HQ)).astype(np.float32)
    p1 = np.zeros((DH, DH), np.float32)
    for m in range(DH // 2):
        p1[2 * m + 1, 2 * m] = -1.0
        p1[2 * m, 2 * m + 1] = 1.0
    pmat = np.kron(np.eye(HQ, dtype=np.float32), p1)
    return cosf, sinf, pmat


_COSF, _SINF, _PMAT = _rope_consts()


def kernel(x, Wq, Wk, Wv, Wo):
    xb = x.reshape(SQ, D).astype(jnp.bfloat16)
    wq = Wq.astype(jnp.bfloat16)
    wk = Wk.astype(jnp.bfloat16)
    wv = Wv.astype(jnp.bfloat16)
    wo = Wo.astype(jnp.bfloat16)
    cosf = jnp.asarray(_COSF)
    sinf = jnp.asarray(_SINF)
    pmat = jnp.asarray(_PMAT, dtype=jnp.bfloat16)

    def body(x_ref, wq_ref, wk_ref, wv_ref, wo_ref, cos_ref, sin_ref, p_ref,
             out_ref, wbuf, rs_rbuf, send_sems, recv_sems):
        my = lax.axis_index("i")

        xv = x_ref[...]
        p = p_ref[...]

        def proj_rope(w_ref):
            t = jnp.dot(xv, w_ref[...], preferred_element_type=jnp.float32)
            tr = jnp.dot(t.astype(jnp.bfloat16), p,
                         preferred_element_type=jnp.float32)
            return (t * cos_ref[...] + tr * sin_ref[...]).astype(jnp.bfloat16)

        q = proj_rope(wq_ref)
        k = proj_rope(wk_ref)
        v = jnp.dot(xv, wv_ref[...],
                    preferred_element_type=jnp.float32).astype(jnp.bfloat16)

        acc = jnp.zeros((SQ, D), jnp.float32)
        for h in range(HQ):
            sl = slice(h * DH, (h + 1) * DH)
            s = lax.dot_general(q[:, sl], k[:, sl],
                                (((1,), (1,)), ((), ())),
                                preferred_element_type=jnp.float32) * SCALE
            m = jnp.max(s, axis=-1, keepdims=True)
            w = jnp.exp(s - m)
            w = w / jnp.sum(w, axis=-1, keepdims=True)
            ctx = jnp.dot(w.astype(jnp.bfloat16), v[:, sl],
                          preferred_element_type=jnp.float32)
            acc = acc + jnp.dot(ctx.astype(jnp.bfloat16), wo_ref[sl, :],
                                preferred_element_type=jnp.float32)
        wbuf[...] = acc.astype(jnp.bfloat16)

        lo = jnp.int32(0)
        roffs = (0, 512, 768)
        for s, mask in enumerate((4, 2, 1)):
            half = CHUNK * mask
            partner = lax.bitwise_xor(my, mask)
            keep_upper = lax.bitwise_and(my, mask) != 0
            send_lo = lo + jnp.where(keep_upper, 0, half)
            new_lo = lo + jnp.where(keep_upper, half, 0)
            rdma = pltpu.make_async_remote_copy(
                src_ref=wbuf.at[pl.ds(send_lo, half), :],
                dst_ref=rs_rbuf.at[pl.ds(roffs[s], half), :],
                send_sem=send_sems.at[s],
                recv_sem=recv_sems.at[s],
                device_id=(partner,),
                device_id_type=pl.DeviceIdType.MESH,
            )
            rdma.start()
            rdma.wait()
            seg = wbuf[pl.ds(new_lo, half), :].astype(jnp.float32)
            inc = rs_rbuf[pl.ds(roffs[s], half), :].astype(jnp.float32)
            wbuf[pl.ds(new_lo, half), :] = (seg + inc).astype(jnp.bfloat16)
            lo = new_lo

        size = CHUNK
        for s, mask in enumerate((1, 2, 4)):
            partner = lax.bitwise_xor(my, mask)
            rdma = pltpu.make_async_remote_copy(
                src_ref=wbuf.at[pl.ds(lo, size), :],
                dst_ref=wbuf.at[pl.ds(lo, size), :],
                send_sem=send_sems.at[3 + s],
                recv_sem=recv_sems.at[3 + s],
                device_id=(partner,),
                device_id_type=pl.DeviceIdType.MESH,
            )
            rdma.start()
            rdma.wait()
            lo = lo - jnp.where(lax.bitwise_and(my, mask) != 0,
                                CHUNK * mask, 0)
            size *= 2

        out_ref[0] = wbuf[...].astype(jnp.float32)

    return pl.pallas_call(
        body,
        out_shape=jax.ShapeDtypeStruct((B, SQ, D), jnp.float32),
        in_specs=[pl.BlockSpec(memory_space=pltpu.VMEM)] * 8,
        out_specs=pl.BlockSpec(memory_space=pltpu.VMEM),
        scratch_shapes=[
            pltpu.VMEM((SQ, D), jnp.bfloat16),
            pltpu.VMEM((7 * CHUNK, D), jnp.bfloat16),
            pltpu.SemaphoreType.DMA((6,)),
            pltpu.SemaphoreType.DMA((6,)),
        ],
    )(xb, wq, wk, wv, wo, cosf, sinf, pmat)


# device time: 98465 ns/iter; 1.6778x vs baseline; 1.1245x over previous
import numpy as np
import jax
import jax.numpy as jnp
from jax import lax
from jax.experimental import pallas as pl
from jax.experimental.pallas import tpu as pltpu

N_DEV = 8
B, SQ, D = 1, 1024, 1024
HQ, DH = 8, 128
CHUNK = SQ // N_DEV
SCALE = 0.08838834764831843


def _rope_consts():
    inv = 1.0 / (10000.0 ** (np.arange(0, DH, 2) / DH))
    pos = np.arange(SQ)[:, None] * inv[None, :]
    cos = np.repeat(np.cos(pos), 2, axis=-1)
    sin = np.repeat(np.sin(pos), 2, axis=-1)
    cosf = np.tile(cos, (1, HQ)).astype(np.float32)
    sinf = np.tile(sin, (1, HQ)).astype(np.float32)
    p1 = np.zeros((DH, DH), np.float32)
    for m in range(DH // 2):
        p1[2 * m + 1, 2 * m] = -1.0
        p1[2 * m, 2 * m + 1] = 1.0
    pmat = np.kron(np.eye(HQ, dtype=np.float32), p1)
    return cosf, sinf, pmat


_COSF, _SINF, _PMAT = _rope_consts()


def kernel(x, Wq, Wk, Wv, Wo):
    xb = x.reshape(SQ, D).astype(jnp.bfloat16)
    wq = Wq.astype(jnp.bfloat16)
    wk = Wk.astype(jnp.bfloat16)
    wv = Wv.astype(jnp.bfloat16)
    wo = Wo.astype(jnp.bfloat16)
    cosf = jnp.asarray(_COSF)
    sinf = jnp.asarray(_SINF)
    pmat = jnp.asarray(_PMAT, dtype=jnp.bfloat16)

    def body(x_ref, wq_ref, wk_ref, wv_ref, wo_ref, cos_ref, sin_ref, p_ref,
             out_ref, wbuf, rs_rbuf, send_sems, recv_sems):
        my = lax.axis_index("i")

        xv = x_ref[...]
        p = p_ref[...]

        def proj_rope(w_ref):
            t = jnp.dot(xv, w_ref[...], preferred_element_type=jnp.float32)
            tr = jnp.dot(t.astype(jnp.bfloat16), p,
                         preferred_element_type=jnp.float32)
            return (t * cos_ref[...] + tr * sin_ref[...]).astype(jnp.bfloat16)

        q = proj_rope(wq_ref)
        k = proj_rope(wk_ref)
        v = jnp.dot(xv, wv_ref[...],
                    preferred_element_type=jnp.float32).astype(jnp.bfloat16)
        q = (q.astype(jnp.float32) * SCALE).astype(jnp.bfloat16)

        ctxs = []
        for h in range(HQ):
            sl = slice(h * DH, (h + 1) * DH)
            s = lax.dot_general(q[:, sl], k[:, sl],
                                (((1,), (1,)), ((), ())),
                                preferred_element_type=jnp.float32)
            w = jnp.exp(s)
            denom = jnp.sum(w, axis=-1, keepdims=True)
            ctx = jnp.dot(w.astype(jnp.bfloat16), v[:, sl],
                          preferred_element_type=jnp.float32)
            ctxs.append((ctx / denom).astype(jnp.bfloat16))
        ctx_full = jnp.concatenate(ctxs, axis=1)
        acc = jnp.dot(ctx_full, wo_ref[...],
                      preferred_element_type=jnp.float32)
        wbuf[...] = acc.astype(jnp.bfloat16)

        lo = jnp.int32(0)
        roffs = (0, 512, 768)
        pending = []
        for s, mask in enumerate((4, 2, 1)):
            half = CHUNK * mask
            partner = lax.bitwise_xor(my, mask)
            keep_upper = lax.bitwise_and(my, mask) != 0
            send_lo = lo + jnp.where(keep_upper, 0, half)
            new_lo = lo + jnp.where(keep_upper, half, 0)
            rdma = pltpu.make_async_remote_copy(
                src_ref=wbuf.at[pl.ds(send_lo, half), :],
                dst_ref=rs_rbuf.at[pl.ds(roffs[s], half), :],
                send_sem=send_sems.at[s],
                recv_sem=recv_sems.at[s],
                device_id=(partner,),
                device_id_type=pl.DeviceIdType.MESH,
            )
            rdma.start()
            rdma.wait_recv()
            pending.append(rdma)
            seg = wbuf[pl.ds(new_lo, half), :].astype(jnp.float32)
            inc = rs_rbuf[pl.ds(roffs[s], half), :].astype(jnp.float32)
            wbuf[pl.ds(new_lo, half), :] = (seg + inc).astype(jnp.bfloat16)
            lo = new_lo

        size = CHUNK
        for s, mask in enumerate((1, 2, 4)):
            partner = lax.bitwise_xor(my, mask)
            rdma = pltpu.make_async_remote_copy(
                src_ref=wbuf.at[pl.ds(lo, size), :],
                dst_ref=wbuf.at[pl.ds(lo, size), :],
                send_sem=send_sems.at[3 + s],
                recv_sem=recv_sems.at[3 + s],
                device_id=(partner,),
                device_id_type=pl.DeviceIdType.MESH,
            )
            rdma.start()
            rdma.wait_recv()
            pending.append(rdma)
            lo = lo - jnp.where(lax.bitwise_and(my, mask) != 0,
                                CHUNK * mask, 0)
            size *= 2

        out_ref[0] = wbuf[...].astype(jnp.float32)
        for rdma in pending:
            rdma.wait_send()

    return pl.pallas_call(
        body,
        out_shape=jax.ShapeDtypeStruct((B, SQ, D), jnp.float32),
        in_specs=[pl.BlockSpec(memory_space=pltpu.VMEM)] * 8,
        out_specs=pl.BlockSpec(memory_space=pltpu.VMEM),
        scratch_shapes=[
            pltpu.VMEM((SQ, D), jnp.bfloat16),
            pltpu.VMEM((7 * CHUNK, D), jnp.bfloat16),
            pltpu.SemaphoreType.DMA((6,)),
            pltpu.SemaphoreType.DMA((6,)),
        ],
    )(xb, wq, wk, wv, wo, cosf, sinf, pmat)


# device time: 85186 ns/iter; 1.9393x vs baseline; 1.1559x over previous
import numpy as np
import jax
import jax.numpy as jnp
from jax import lax
from jax.experimental import pallas as pl
from jax.experimental.pallas import tpu as pltpu

N_DEV = 8
B, SQ, D = 1, 1024, 1024
HQ, DH = 8, 128
CHUNK = SQ // N_DEV
SCALE = 0.08838834764831843


def _rope_consts():
    inv = 1.0 / (10000.0 ** (np.arange(0, DH, 2) / DH))
    pos = np.arange(SQ)[:, None] * inv[None, :]
    cos = np.repeat(np.cos(pos), 2, axis=-1)
    sin = np.repeat(np.sin(pos), 2, axis=-1)
    cosf = np.tile(cos, (1, HQ)).astype(np.float32)
    sinf = np.tile(sin, (1, HQ)).astype(np.float32)
    p1 = np.zeros((DH, DH), np.float32)
    for m in range(DH // 2):
        p1[2 * m + 1, 2 * m] = -1.0
        p1[2 * m, 2 * m + 1] = 1.0
    pmat = np.kron(np.eye(HQ, dtype=np.float32), p1)
    return cosf, sinf, pmat


_COSF, _SINF, _PMAT = _rope_consts()


def kernel(x, Wq, Wk, Wv, Wo):
    xb = x.reshape(SQ, D).astype(jnp.bfloat16)
    wq = Wq.astype(jnp.bfloat16)
    wk = Wk.astype(jnp.bfloat16)
    wv = Wv.astype(jnp.bfloat16)
    wo = Wo.astype(jnp.bfloat16)
    cosf = jnp.asarray(_COSF)
    sinf = jnp.asarray(_SINF)
    pmat = jnp.asarray(_PMAT, dtype=jnp.bfloat16)

    def body(x_ref, wq_ref, wk_ref, wv_ref, wo_ref, cos_ref, sin_ref, p_ref,
             out_ref, wbufA, wbufB, rbufA, rbufB, send_sems, recv_sems):
        my = lax.axis_index("i")

        xv = x_ref[...]
        p = p_ref[...]

        def proj_rope(w_ref):
            t = jnp.dot(xv, w_ref[...], preferred_element_type=jnp.float32)
            tr = jnp.dot(t.astype(jnp.bfloat16), p,
                         preferred_element_type=jnp.float32)
            return (t * cos_ref[...] + tr * sin_ref[...]).astype(jnp.bfloat16)

        q = proj_rope(wq_ref)
        k = proj_rope(wk_ref)
        v = jnp.dot(xv, wv_ref[...],
                    preferred_element_type=jnp.float32).astype(jnp.bfloat16)
        q = (q.astype(jnp.float32) * SCALE).astype(jnp.bfloat16)

        ctxs = []
        for h in range(HQ):
            sl = slice(h * DH, (h + 1) * DH)
            s = lax.dot_general(q[:, sl], k[:, sl],
                                (((1,), (1,)), ((), ())),
                                preferred_element_type=jnp.float32)
            w = jnp.exp(s)
            denom = jnp.sum(w, axis=-1, keepdims=True)
            ctx = jnp.dot(w.astype(jnp.bfloat16), v[:, sl],
                          preferred_element_type=jnp.float32)
            ctxs.append((ctx / denom).astype(jnp.bfloat16))
        ctx_full = jnp.concatenate(ctxs, axis=1)
        acc = jnp.dot(ctx_full, wo_ref[...],
                      preferred_element_type=jnp.float32)
        wbufA[...] = acc[:, :512].astype(jnp.bfloat16)
        wbufB[...] = acc[:, 512:].astype(jnp.bfloat16)

        SLABS = ((wbufA, rbufA, (4, 2, 1)), (wbufB, rbufB, (1, 2, 4)))
        los = [jnp.int32(0), jnp.int32(0)]
        roffs = (0, 512, 768)
        pending = []
        for s in range(3):
            half = 512 >> s
            rdmas = []
            for j, (wb, rb, masks) in enumerate(SLABS):
                mask = masks[s]
                partner = lax.bitwise_xor(my, mask)
                keep_upper = lax.bitwise_and(my, mask) != 0
                send_lo = los[j] + jnp.where(keep_upper, 0, half)
                los[j] = los[j] + jnp.where(keep_upper, half, 0)
                rdma = pltpu.make_async_remote_copy(
                    src_ref=wb.at[pl.ds(send_lo, half), :],
                    dst_ref=rb.at[pl.ds(roffs[s], half), :],
                    send_sem=send_sems.at[2 * s + j],
                    recv_sem=recv_sems.at[2 * s + j],
                    device_id=(partner,),
                    device_id_type=pl.DeviceIdType.MESH,
                )
                rdma.start()
                rdmas.append(rdma)
            for j, (wb, rb, masks) in enumerate(SLABS):
                rdmas[j].wait_recv()
                pending.append(rdmas[j])
                seg = wb[pl.ds(los[j], half), :].astype(jnp.float32)
                inc = rb[pl.ds(roffs[s], half), :].astype(jnp.float32)
                wb[pl.ds(los[j], half), :] = (seg + inc).astype(jnp.bfloat16)

        for s in range(3):
            size = CHUNK << s
            rdmas = []
            for j, (wb, rb, masks) in enumerate(SLABS):
                mask = masks[2 - s]
                partner = lax.bitwise_xor(my, mask)
                rdma = pltpu.make_async_remote_copy(
                    src_ref=wb.at[pl.ds(los[j], size), :],
                    dst_ref=wb.at[pl.ds(los[j], size), :],
                    send_sem=send_sems.at[6 + 2 * s + j],
                    recv_sem=recv_sems.at[6 + 2 * s + j],
                    device_id=(partner,),
                    device_id_type=pl.DeviceIdType.MESH,
                )
                rdma.start()
                rdmas.append(rdma)
            for j, (wb, rb, masks) in enumerate(SLABS):
                mask = masks[2 - s]
                rdmas[j].wait_recv()
                pending.append(rdmas[j])
                los[j] = los[j] - jnp.where(
                    lax.bitwise_and(my, mask) != 0, size, 0)

        out_ref[0, :, :512] = wbufA[...].astype(jnp.float32)
        out_ref[0, :, 512:] = wbufB[...].astype(jnp.float32)
        for rdma in pending:
            rdma.wait_send()

    return pl.pallas_call(
        body,
        out_shape=jax.ShapeDtypeStruct((B, SQ, D), jnp.float32),
        in_specs=[pl.BlockSpec(memory_space=pltpu.VMEM)] * 8,
        out_specs=pl.BlockSpec(memory_space=pltpu.VMEM),
        scratch_shapes=[
            pltpu.VMEM((SQ, D // 2), jnp.bfloat16),
            pltpu.VMEM((SQ, D // 2), jnp.bfloat16),
            pltpu.VMEM((7 * CHUNK, D // 2), jnp.bfloat16),
            pltpu.VMEM((7 * CHUNK, D // 2), jnp.bfloat16),
            pltpu.SemaphoreType.DMA((12,)),
            pltpu.SemaphoreType.DMA((12,)),
        ],
    )(xb, wq, wk, wv, wo, cosf, sinf, pmat)
